# Initial kernel scaffold; baseline (speedup 1.0000x reference)
import jax
import jax.numpy as jnp
from jax.experimental import pallas as pl


def kernel(hidden, edge_index, W_h, W_hf):
    raise NotImplementedError("write your pallas kernel here")


# speedup vs baseline: 2.5368x; 2.5368x over previous
import jax
import jax.numpy as jnp
from jax import lax
from jax.experimental import pallas as pl
from jax.experimental.pallas import tpu as pltpu
from jax.experimental.pallas import tpu_sc as plsc

N = 10000
E = 320000
D = 128
NP = 10240
NC = 2
NS = 16
NW = NC * NS
CH = 128
CPT = 80
TOT = NW * CPT
E_PAD = TOT * CH
RPS = NP // NS
HALF = RPS // 2

_mesh = plsc.VectorSubcoreMesh(core_axis_name="c", subcore_axis_name="s")


def _deg_body(srcc, dstc, out, src_idx, dst_idx, ones_v, zbuf, deg_s_sh, deg_d_sh):
    cid = lax.axis_index("c")
    sid = lax.axis_index("s")
    wid = sid * NC + cid
    for i in range(RPS // 16):
        zbuf[pl.ds(i * 16, 16)] = jnp.zeros((16,), jnp.float32)
    for i in range(CH // 16):
        ones_v[pl.ds(i * 16, 16)] = jnp.ones((16,), jnp.float32)
    pltpu.sync_copy(zbuf, deg_s_sh.at[pl.ds(sid * RPS, RPS)])
    pltpu.sync_copy(zbuf, deg_d_sh.at[pl.ds(sid * RPS, RPS)])
    plsc.subcore_barrier()
    pltpu.sync_copy(srcc.at[pl.ds(wid * CPT, CPT)], src_idx)
    pltpu.sync_copy(dstc.at[pl.ds(wid * CPT, CPT)], dst_idx)

    def body(j, c):
        pltpu.sync_copy(ones_v, deg_s_sh.at[src_idx.at[j]], add=True)
        pltpu.sync_copy(ones_v, deg_d_sh.at[dst_idx.at[j]], add=True)
        return c

    lax.fori_loop(0, CPT, body, 0)
    plsc.subcore_barrier()
    pltpu.sync_copy(deg_s_sh.at[pl.ds(sid * RPS, RPS)], zbuf)
    pltpu.sync_copy(zbuf, out.at[cid, 0, pl.ds(sid * RPS, RPS)])
    pltpu.sync_copy(deg_d_sh.at[pl.ds(sid * RPS, RPS)], zbuf)
    pltpu.sync_copy(zbuf, out.at[cid, 1, pl.ds(sid * RPS, RPS)])


_deg_call = pl.kernel(
    _deg_body,
    out_type=jax.ShapeDtypeStruct((NC, 2, NP), jnp.float32),
    mesh=_mesh,
    scratch_types=[
        pltpu.VMEM((CPT, CH), jnp.int32),
        pltpu.VMEM((CPT, CH), jnp.int32),
        pltpu.VMEM((CH,), jnp.float32),
        pltpu.VMEM((RPS,), jnp.float32),
        pltpu.VMEM_SHARED((NP,), jnp.float32),
        pltpu.VMEM_SHARED((NP,), jnp.float32),
    ],
)


def _agg_body(feat, srcc, dstc, out, src_idx, dst_idx, gbuf, accum_sh):
    cid = lax.axis_index("c")
    sid = lax.axis_index("s")
    wid = sid * NC + cid

    def zrow(i, c):
        for k in range(D // 16):
            gbuf[i, pl.ds(k * 16, 16)] = jnp.zeros((16,), jnp.float32)
        return c

    lax.fori_loop(0, CH, zrow, 0)
    for z in range(RPS // CH):
        pltpu.sync_copy(gbuf, accum_sh.at[pl.ds(sid * RPS + z * CH, CH)])
    plsc.subcore_barrier()
    pltpu.sync_copy(srcc.at[pl.ds(wid * CPT, CPT)], src_idx)
    pltpu.sync_copy(dstc.at[pl.ds(wid * CPT, CPT)], dst_idx)

    def body(j, c):
        pltpu.sync_copy(feat.at[src_idx.at[j]], gbuf)
        pltpu.sync_copy(gbuf, accum_sh.at[dst_idx.at[j]], add=True)
        return c

    lax.fori_loop(0, CPT, body, 0)
    plsc.subcore_barrier()
    for z in range(RPS // CH):
        pltpu.sync_copy(accum_sh.at[pl.ds(sid * RPS + z * CH, CH)], gbuf)
        pltpu.sync_copy(gbuf, out.at[cid, pl.ds(sid * RPS + z * CH, CH)])


_agg_call = pl.kernel(
    _agg_body,
    out_type=jax.ShapeDtypeStruct((NC, NP, D), jnp.float32),
    mesh=_mesh,
    scratch_types=[
        pltpu.VMEM((CPT, CH), jnp.int32),
        pltpu.VMEM((CPT, CH), jnp.int32),
        pltpu.VMEM((CH, D), jnp.float32),
        pltpu.VMEM_SHARED((NP, D), jnp.float32),
    ],
)


R2 = 1280


def _mm1_body(d_ref, h_ref, w_ref, feat_ref, h2_ref, h3_ref):
    d = d_ref[...]
    dsrc = d[:, 0:1] + d[:, 1:2]
    nsrc = lax.rsqrt(jnp.maximum(dsrc, 1.0))
    hw = jnp.dot(h_ref[...], w_ref[...], preferred_element_type=jnp.float32)
    feat_ref[...] = hw[:, :D] * nsrc
    h2_ref[...] = hw[:, D:2 * D]
    h3_ref[...] = hw[:, 2 * D:]


_mm1_call = pl.pallas_call(
    _mm1_body,
    grid=(NP // R2,),
    in_specs=[
        pl.BlockSpec((R2, 4), lambda i: (i, 0)),
        pl.BlockSpec((R2, D), lambda i: (i, 0)),
        pl.BlockSpec((D, 3 * D), lambda i: (0, 0)),
    ],
    out_specs=[pl.BlockSpec((R2, D), lambda i: (i, 0))] * 3,
    out_shape=[jax.ShapeDtypeStruct((NP, D), jnp.float32)] * 3,
)


def _mm2_body(d_ref, p_ref, h2_ref, h3_ref, w_ref, o_ref):
    d = d_ref[...]
    ddst = d[:, 2:3] + d[:, 3:4]
    ndst = lax.rsqrt(jnp.maximum(ddst, 1.0))
    p = p_ref[...]
    agg = (p[0] + p[1]) * ndst
    hf = jnp.dot(agg, w_ref[...], preferred_element_type=jnp.float32)
    gate = jnp.maximum(hf[:, :D] + h2_ref[...], 0.0)
    o_ref[...] = h3_ref[...] + gate * hf[:, D:]


_mm2_call = pl.pallas_call(
    _mm2_body,
    grid=(NP // R2,),
    in_specs=[
        pl.BlockSpec((R2, 4), lambda i: (i, 0)),
        pl.BlockSpec((NC, R2, D), lambda i: (0, i, 0)),
        pl.BlockSpec((R2, D), lambda i: (i, 0)),
        pl.BlockSpec((R2, D), lambda i: (i, 0)),
        pl.BlockSpec((D, 2 * D), lambda i: (0, 0)),
    ],
    out_specs=pl.BlockSpec((R2, D), lambda i: (i, 0)),
    out_shape=jax.ShapeDtypeStruct((NP, D), jnp.float32),
)


def kernel(hidden, edge_index, W_h, W_hf):
    src = edge_index[0]
    dst = edge_index[1]
    pad = E_PAD - E
    srcc = jnp.concatenate([src, jnp.full((pad,), N, jnp.int32)]).reshape(TOT, CH)
    dstc = jnp.concatenate([dst, jnp.full((pad,), N, jnp.int32)]).reshape(TOT, CH)
    h_pad = jnp.pad(hidden, ((0, NP - N), (0, 0)))

    degs = _deg_call(srcc, dstc)
    degt = jnp.transpose(degs, (2, 1, 0)).reshape(NP, 4)
    feat, h2, h3 = _mm1_call(degt, h_pad, W_h)
    parts = _agg_call(feat, srcc, dstc)
    out = _mm2_call(degt, parts, h2, h3, W_hf)
    res = out[:N]
    return (res, res)


# speedup vs baseline: 2.7520x; 1.0849x over previous
import jax
import jax.numpy as jnp
from jax import lax
from jax.experimental import pallas as pl
from jax.experimental.pallas import tpu as pltpu
from jax.experimental.pallas import tpu_sc as plsc

N = 10000
E = 320000
D = 128
NP = 10240
NC = 2
NS = 16
NW = NC * NS
CH = 128
CPT = 80
TOT = NW * CPT
E_PAD = TOT * CH
RPS = NP // NS
HALF = RPS // 2

_mesh = plsc.VectorSubcoreMesh(core_axis_name="c", subcore_axis_name="s")


def _deg_body(srcc, dstc, out, src_idx, dst_idx, ones_v, zbuf, deg_s_sh, deg_d_sh):
    cid = lax.axis_index("c")
    sid = lax.axis_index("s")
    wid = sid * NC + cid
    for i in range(RPS // 16):
        zbuf[pl.ds(i * 16, 16)] = jnp.zeros((16,), jnp.float32)
    for i in range(CH // 16):
        ones_v[pl.ds(i * 16, 16)] = jnp.ones((16,), jnp.float32)
    pltpu.sync_copy(zbuf, deg_s_sh.at[pl.ds(sid * RPS, RPS)])
    pltpu.sync_copy(zbuf, deg_d_sh.at[pl.ds(sid * RPS, RPS)])
    plsc.subcore_barrier()
    pltpu.sync_copy(srcc.at[pl.ds(wid * CPT, CPT)], src_idx)
    pltpu.sync_copy(dstc.at[pl.ds(wid * CPT, CPT)], dst_idx)

    def body(j, c):
        pltpu.sync_copy(ones_v, deg_s_sh.at[src_idx.at[j]], add=True)
        pltpu.sync_copy(ones_v, deg_d_sh.at[dst_idx.at[j]], add=True)
        return c

    lax.fori_loop(0, CPT, body, 0)
    plsc.subcore_barrier()
    pltpu.sync_copy(deg_s_sh.at[pl.ds(sid * RPS, RPS)], zbuf)
    pltpu.sync_copy(zbuf, out.at[cid, 0, pl.ds(sid * RPS, RPS)])
    pltpu.sync_copy(deg_d_sh.at[pl.ds(sid * RPS, RPS)], zbuf)
    pltpu.sync_copy(zbuf, out.at[cid, 1, pl.ds(sid * RPS, RPS)])


_deg_call = pl.kernel(
    _deg_body,
    out_type=jax.ShapeDtypeStruct((NC, 2, NP), jnp.float32),
    mesh=_mesh,
    scratch_types=[
        pltpu.VMEM((CPT, CH), jnp.int32),
        pltpu.VMEM((CPT, CH), jnp.int32),
        pltpu.VMEM((CH,), jnp.float32),
        pltpu.VMEM((RPS,), jnp.float32),
        pltpu.VMEM_SHARED((NP,), jnp.float32),
        pltpu.VMEM_SHARED((NP,), jnp.float32),
    ],
)


HCH = CPT // 2


def _agg_body(feat, srcc, dstc, out, src_idx, dst_idx, gbuf0, gbuf1, sem0, sem1,
              accum_sh):
    cid = lax.axis_index("c")
    sid = lax.axis_index("s")
    wid = sid * NC + cid

    def zrow(i, c):
        for k in range(D // 16):
            gbuf0[i, pl.ds(k * 16, 16)] = jnp.zeros((16,), jnp.float32)
        return c

    lax.fori_loop(0, CH, zrow, 0)
    for z in range(RPS // CH):
        pltpu.sync_copy(gbuf0, accum_sh.at[pl.ds(sid * RPS + z * CH, CH)])
    plsc.subcore_barrier()

    for h in range(2):
        base = wid * CPT + h * HCH
        pltpu.sync_copy(srcc.at[pl.ds(base, HCH)], src_idx)
        pltpu.sync_copy(dstc.at[pl.ds(base, HCH)], dst_idx)
        pltpu.async_copy(feat.at[src_idx.at[0]], gbuf0, sem0)

        def pair(jj, c):
            j = jj * 2
            pltpu.async_copy(feat.at[src_idx.at[j + 1]], gbuf1, sem1)
            pltpu.make_async_copy(feat.at[src_idx.at[j]], gbuf0, sem0).wait()
            pltpu.sync_copy(gbuf0, accum_sh.at[dst_idx.at[j]], add=True)
            nxt = jnp.minimum(j + 2, HCH - 1)
            pltpu.async_copy(feat.at[src_idx.at[nxt]], gbuf0, sem0)
            pltpu.make_async_copy(feat.at[src_idx.at[j + 1]], gbuf1, sem1).wait()
            pltpu.sync_copy(gbuf1, accum_sh.at[dst_idx.at[j + 1]], add=True)
            return c

        lax.fori_loop(0, HCH // 2, pair, 0)
        pltpu.make_async_copy(feat.at[src_idx.at[HCH - 1]], gbuf0, sem0).wait()

    plsc.subcore_barrier()
    for z in range(RPS // CH):
        pltpu.sync_copy(accum_sh.at[pl.ds(sid * RPS + z * CH, CH)], gbuf0)
        pltpu.sync_copy(gbuf0, out.at[cid, pl.ds(sid * RPS + z * CH, CH)])


_agg_call = pl.kernel(
    _agg_body,
    out_type=jax.ShapeDtypeStruct((NC, NP, D), jnp.float32),
    mesh=_mesh,
    scratch_types=[
        pltpu.VMEM((HCH, CH), jnp.int32),
        pltpu.VMEM((HCH, CH), jnp.int32),
        pltpu.VMEM((CH, D), jnp.float32),
        pltpu.VMEM((CH, D), jnp.float32),
        pltpu.SemaphoreType.DMA,
        pltpu.SemaphoreType.DMA,
        pltpu.VMEM_SHARED((NP, D), jnp.float32),
    ],
)


R2 = 1280


def _mm1_body(d_ref, h_ref, w_ref, feat_ref, h2_ref, h3_ref):
    d = d_ref[...]
    dsrc = d[:, 0:1] + d[:, 1:2]
    nsrc = lax.rsqrt(jnp.maximum(dsrc, 1.0))
    hw = jnp.dot(h_ref[...], w_ref[...], preferred_element_type=jnp.float32)
    feat_ref[...] = hw[:, :D] * nsrc
    h2_ref[...] = hw[:, D:2 * D]
    h3_ref[...] = hw[:, 2 * D:]


_mm1_call = pl.pallas_call(
    _mm1_body,
    grid=(NP // R2,),
    in_specs=[
        pl.BlockSpec((R2, 4), lambda i: (i, 0)),
        pl.BlockSpec((R2, D), lambda i: (i, 0)),
        pl.BlockSpec((D, 3 * D), lambda i: (0, 0)),
    ],
    out_specs=[pl.BlockSpec((R2, D), lambda i: (i, 0))] * 3,
    out_shape=[jax.ShapeDtypeStruct((NP, D), jnp.float32)] * 3,
)


def _mm2_body(d_ref, p_ref, h2_ref, h3_ref, w_ref, o_ref):
    d = d_ref[...]
    ddst = d[:, 2:3] + d[:, 3:4]
    ndst = lax.rsqrt(jnp.maximum(ddst, 1.0))
    p = p_ref[...]
    agg = (p[0] + p[1]) * ndst
    hf = jnp.dot(agg, w_ref[...], preferred_element_type=jnp.float32)
    gate = jnp.maximum(hf[:, :D] + h2_ref[...], 0.0)
    o_ref[...] = h3_ref[...] + gate * hf[:, D:]


_mm2_call = pl.pallas_call(
    _mm2_body,
    grid=(NP // R2,),
    in_specs=[
        pl.BlockSpec((R2, 4), lambda i: (i, 0)),
        pl.BlockSpec((NC, R2, D), lambda i: (0, i, 0)),
        pl.BlockSpec((R2, D), lambda i: (i, 0)),
        pl.BlockSpec((R2, D), lambda i: (i, 0)),
        pl.BlockSpec((D, 2 * D), lambda i: (0, 0)),
    ],
    out_specs=pl.BlockSpec((R2, D), lambda i: (i, 0)),
    out_shape=jax.ShapeDtypeStruct((NP, D), jnp.float32),
)


def kernel(hidden, edge_index, W_h, W_hf):
    src = edge_index[0]
    dst = edge_index[1]
    pad = E_PAD - E
    srcc = jnp.concatenate([src, jnp.full((pad,), N, jnp.int32)]).reshape(TOT, CH)
    dstc = jnp.concatenate([dst, jnp.full((pad,), N, jnp.int32)]).reshape(TOT, CH)
    h_pad = jnp.pad(hidden, ((0, NP - N), (0, 0)))

    degs = _deg_call(srcc, dstc)
    degt = jnp.transpose(degs, (2, 1, 0)).reshape(NP, 4)
    feat, h2, h3 = _mm1_call(degt, h_pad, W_h)
    parts = _agg_call(feat, srcc, dstc)
    out = _mm2_call(degt, parts, h2, h3, W_hf)
    res = out[:N]
    return (res, res)
